# baseline (speedup 1.0000x reference)
import functools

import jax
import jax.numpy as jnp
from jax import lax
from jax.experimental import pallas as pl
from jax.experimental.pallas import tpu as pltpu
from jax.experimental.pallas import tpu_sc as plsc

TIME = 4096
FREQ = 513
CHUNK = 20
N_WIN = TIME - CHUNK
WIN_WORDS = FREQ * CHUNK
PAIR_WORDS = 2 * WIN_WORDS
PAIR_PAD = 20544
N_PAIRS = N_WIN // 2
NW = 32
PAIR_BASE = 63
PAIR_EXTRA = N_PAIRS - PAIR_BASE * NW
STAGE_ROWS = 160
STAGE_WORDS = STAGE_ROWS * FREQ
Y_BLOCKS = N_WIN // 8
Y_BASE = 15
Y_EXTRA = Y_BLOCKS - Y_BASE * NW

_mesh = plsc.VectorSubcoreMesh(core_axis_name="c", subcore_axis_name="s")


@functools.partial(
    pl.kernel,
    mesh=_mesh,
    out_type=(
        jax.ShapeDtypeStruct((N_WIN * WIN_WORDS,), jnp.float32),
        jax.ShapeDtypeStruct((N_WIN, FREQ), jnp.float32),
    ),
    scratch_types=[
        pltpu.VMEM((STAGE_WORDS,), jnp.float32),
        pltpu.VMEM((PAIR_PAD,), jnp.int32),
        pltpu.VMEM((PAIR_PAD,), jnp.float32),
    ],
    compiler_params=pltpu.CompilerParams(needs_layout_passes=False),
)
def _sc_chunk(mixed_ref, vocal_ref, p2_ref, x_ref, y_ref, in_v, p2_v, ob_v):
    w = lax.axis_index("s") * 2 + lax.axis_index("c")
    p0 = w * PAIR_BASE + jnp.minimum(w, PAIR_EXTRA)
    row0 = jnp.bitwise_and(2 * p0, -8)
    in_off = pl.multiple_of(row0 * FREQ, 8)
    pltpu.sync_copy(mixed_ref.at[pl.ds(in_off, STAGE_WORDS)], in_v)
    pltpu.sync_copy(p2_ref, p2_v)

    def do_pair(p):
        woff = (2 * p - row0) * FREQ

        def gi(i, carry):
            b = i * 64
            for k in range(4):
                bb = b + k * 16
                idxs = p2_v[pl.ds(bb, 16)] + woff
                ob_v[pl.ds(bb, 16)] = plsc.load_gather(in_v, [idxs])
            return carry

        lax.fori_loop(0, PAIR_PAD // 64, gi, 0)
        pltpu.sync_copy(ob_v.at[pl.ds(0, PAIR_WORDS)],
                        x_ref.at[pl.ds(pl.multiple_of(p * PAIR_WORDS, 8), PAIR_WORDS)])

    lax.fori_loop(0, PAIR_BASE, lambda j, c: (do_pair(p0 + j), c)[1], 0)

    @pl.when(w < PAIR_EXTRA)
    def _():
        do_pair(p0 + PAIR_BASE)

    r0 = 8 * (w * Y_BASE + jnp.minimum(w, Y_EXTRA))
    src0 = pl.multiple_of(CHUNK + 4 + r0, 8)
    dst0 = pl.multiple_of(r0, 8)
    pltpu.sync_copy(vocal_ref.at[pl.ds(src0, 8 * Y_BASE)],
                    y_ref.at[pl.ds(dst0, 8 * Y_BASE)])

    @pl.when(w < Y_EXTRA)
    def _():
        pltpu.sync_copy(vocal_ref.at[pl.ds(src0 + 8 * Y_BASE, 8)],
                        y_ref.at[pl.ds(dst0 + 8 * Y_BASE, 8)])

    @pl.when(w == NW - 1)
    def _():
        pltpu.sync_copy(vocal_ref.at[pl.ds(TIME + 4 - 4, 4)],
                        y_ref.at[pl.ds(N_WIN - 4, 4)])


def kernel(mixed_mag, vocal_mag):
    mixed_flat = jnp.pad(mixed_mag, ((0, 16), (0, 0))).reshape(-1)
    vocal_pad = jnp.pad(vocal_mag, ((4, 0), (0, 0)))
    k = jnp.arange(WIN_WORDS, dtype=jnp.int32)
    p1 = (k % CHUNK) * FREQ + (k // CHUNK)
    p2 = jnp.concatenate([p1, p1 + FREQ, jnp.zeros(PAIR_PAD - PAIR_WORDS, jnp.int32)])
    x_flat, y = _sc_chunk(mixed_flat, vocal_pad, p2)
    return x_flat.reshape(N_WIN, FREQ, CHUNK), y


# speedup vs baseline: 1.0272x; 1.0272x over previous
import functools

import jax
import jax.numpy as jnp
from jax import lax
from jax.experimental import pallas as pl
from jax.experimental.pallas import tpu as pltpu
from jax.experimental.pallas import tpu_sc as plsc

TIME = 4096
FREQ = 513
CHUNK = 20
N_WIN = TIME - CHUNK
WIN_WORDS = FREQ * CHUNK
PAIR_WORDS = 2 * WIN_WORDS
PAIR_PAD = 20544
N_PAIRS = N_WIN // 2
NW = 32
PAIR_BASE = 63
PAIR_EXTRA = N_PAIRS - PAIR_BASE * NW
STAGE_ROWS = 160
STAGE_WORDS = STAGE_ROWS * FREQ
Y_BLOCKS = N_WIN // 8
Y_BASE = 15
Y_EXTRA = Y_BLOCKS - Y_BASE * NW

_mesh = plsc.VectorSubcoreMesh(core_axis_name="c", subcore_axis_name="s")


@functools.partial(
    pl.kernel,
    mesh=_mesh,
    out_type=(
        jax.ShapeDtypeStruct((N_WIN * WIN_WORDS,), jnp.float32),
        jax.ShapeDtypeStruct((N_WIN, FREQ), jnp.float32),
    ),
    scratch_types=[
        pltpu.VMEM((STAGE_WORDS,), jnp.float32),
        pltpu.VMEM((PAIR_PAD,), jnp.int32),
        pltpu.VMEM((PAIR_PAD,), jnp.float32),
    ],
    compiler_params=pltpu.CompilerParams(needs_layout_passes=False),
)
def _sc_chunk(mixed_ref, vocal_ref, p2_ref, x_ref, y_ref, in_v, p2_v, ob_v):
    w = lax.axis_index("s") * 2 + lax.axis_index("c")
    p0 = w * PAIR_BASE + jnp.minimum(w, PAIR_EXTRA)
    row0 = jnp.bitwise_and(2 * p0, -8)
    in_off = pl.multiple_of(row0 * FREQ, 8)
    pltpu.sync_copy(mixed_ref.at[pl.ds(in_off, STAGE_WORDS)], in_v)
    pltpu.sync_copy(p2_ref, p2_v)

    def do_pair(p):
        woff = (2 * p - row0) * FREQ

        @plsc.parallel_loop(0, PAIR_PAD, 16, unroll=8)
        def _(bb):
            idxs = p2_v[pl.ds(bb, 16)] + woff
            ob_v[pl.ds(bb, 16)] = plsc.load_gather(in_v, [idxs])
        pltpu.sync_copy(ob_v.at[pl.ds(0, PAIR_WORDS)],
                        x_ref.at[pl.ds(pl.multiple_of(p * PAIR_WORDS, 8), PAIR_WORDS)])

    lax.fori_loop(0, PAIR_BASE, lambda j, c: (do_pair(p0 + j), c)[1], 0)

    @pl.when(w < PAIR_EXTRA)
    def _():
        do_pair(p0 + PAIR_BASE)

    r0 = 8 * (w * Y_BASE + jnp.minimum(w, Y_EXTRA))
    src0 = pl.multiple_of(CHUNK + 4 + r0, 8)
    dst0 = pl.multiple_of(r0, 8)
    pltpu.sync_copy(vocal_ref.at[pl.ds(src0, 8 * Y_BASE)],
                    y_ref.at[pl.ds(dst0, 8 * Y_BASE)])

    @pl.when(w < Y_EXTRA)
    def _():
        pltpu.sync_copy(vocal_ref.at[pl.ds(src0 + 8 * Y_BASE, 8)],
                        y_ref.at[pl.ds(dst0 + 8 * Y_BASE, 8)])

    @pl.when(w == NW - 1)
    def _():
        pltpu.sync_copy(vocal_ref.at[pl.ds(TIME + 4 - 4, 4)],
                        y_ref.at[pl.ds(N_WIN - 4, 4)])


def kernel(mixed_mag, vocal_mag):
    mixed_flat = jnp.pad(mixed_mag, ((0, 16), (0, 0))).reshape(-1)
    vocal_pad = jnp.pad(vocal_mag, ((4, 0), (0, 0)))
    k = jnp.arange(WIN_WORDS, dtype=jnp.int32)
    p1 = (k % CHUNK) * FREQ + (k // CHUNK)
    p2 = jnp.concatenate([p1, p1 + FREQ, jnp.zeros(PAIR_PAD - PAIR_WORDS, jnp.int32)])
    x_flat, y = _sc_chunk(mixed_flat, vocal_pad, p2)
    return x_flat.reshape(N_WIN, FREQ, CHUNK), y


# speedup vs baseline: 122.2705x; 119.0304x over previous
import functools

import jax
import jax.numpy as jnp
from jax.experimental import pallas as pl
from jax.experimental.pallas import tpu as pltpu

TIME = 4096
FREQ = 513
CHUNK = 20
N_WIN = TIME - CHUNK
JB = 256
NJ = (N_WIN + JB - 1) // JB
PADW = TIME + 128


def _body(mt_hbm, vt_hbm, x_ref, y_ref, mscr, vscr, sem0, sem1):
    jb = pl.program_id(0)

    @pl.when(jb == 0)
    def _():
        cp0 = pltpu.make_async_copy(mt_hbm, mscr.at[:, pl.ds(0, TIME)], sem0)
        cp1 = pltpu.make_async_copy(vt_hbm, vscr.at[:, pl.ds(0, TIME)], sem1)
        cp0.start()
        cp1.start()
        cp0.wait()
        cp1.wait()

    base = pl.multiple_of(jb * JB, 128)
    w = mscr[:, pl.ds(base, JB + 128)]
    for c in range(CHUNK):
        x_ref[c, :, :] = w[:, c:c + JB]
    wv = vscr[:, pl.ds(base, JB + 128)]
    y_ref[...] = wv[:, CHUNK:CHUNK + JB]


_call = pl.pallas_call(
    _body,
    grid=(NJ,),
    in_specs=[
        pl.BlockSpec(memory_space=pl.ANY),
        pl.BlockSpec(memory_space=pl.ANY),
    ],
    out_specs=[
        pl.BlockSpec((CHUNK, FREQ, JB), lambda j: (0, 0, j)),
        pl.BlockSpec((FREQ, JB), lambda j: (0, j)),
    ],
    out_shape=[
        jax.ShapeDtypeStruct((CHUNK, FREQ, N_WIN), jnp.float32),
        jax.ShapeDtypeStruct((FREQ, N_WIN), jnp.float32),
    ],
    scratch_shapes=[
        pltpu.VMEM((FREQ, PADW), jnp.float32),
        pltpu.VMEM((FREQ, PADW), jnp.float32),
        pltpu.SemaphoreType.DMA,
        pltpu.SemaphoreType.DMA,
    ],
    compiler_params=pltpu.CompilerParams(vmem_limit_bytes=58 * 1024 * 1024),
)


def kernel(mixed_mag, vocal_mag):
    mt = mixed_mag.T
    vt = vocal_mag.T
    x_alt, y_alt = _call(mt, vt)
    return x_alt.transpose(2, 1, 0), y_alt.T
